# baseline (speedup 1.0000x reference)
import jax
import jax.numpy as jnp
from jax import lax
from jax.experimental import pallas as pl
from jax.experimental.pallas import tpu as pltpu
from jax.experimental.pallas import tpu_sc as plsc
import functools

_NC = 2
_NS = 16
_NW = _NC * _NS
_XCH = 16
_PCH = 16


def _sc_body(batch, seq, hid, x_hbm, pos_hbm, out_hbm,
             bufx, bufp, sx0, sx1, sx2, sx3, sx4, sp0, sp1, so0, so1, so2, so3, so4):
    cid = lax.axis_index("c")
    sid = lax.axis_index("s")
    wid = sid * _NC + cid
    span = seq // _NW
    pstart = wid * span
    npos = span // _PCH
    xc_per_group = (_PCH // _XCH) * batch
    nx = npos * xc_per_group
    per_row = hid // 16
    nvec = _XCH * per_row
    sx = (sx0, sx1, sx2, sx3, sx4)
    sp = (sp0, sp1)
    so = (so0, so1, so2, so3, so4)

    def xrow(g):
        p, r = divmod(g, xc_per_group)
        h, b = divmod(r, batch)
        return b * seq + pstart + p * _PCH + h * _XCH, h

    x_cp = [None] * nx
    p_cp = [None] * npos
    out_cp = [None] * nx

    def load_x(g):
        row, _ = xrow(g)
        x_cp[g] = pltpu.async_copy(
            x_hbm.at[pl.ds(row, _XCH)], bufx.at[g % 5], sx[g % 5])

    def load_p(p):
        p_cp[p] = pltpu.async_copy(
            pos_hbm.at[pl.ds(pstart + p * _PCH, _PCH)], bufp.at[p & 1],
            sp[p & 1])

    load_p(0)
    if npos > 1:
        load_p(1)
    for g0 in range(min(4, nx)):
        load_x(g0)
    for g in range(nx):
        s = g % 5
        p, r = divmod(g, xc_per_group)
        if g + 4 < nx:
            if g >= 1:
                out_cp[g - 1].wait()
            load_x(g + 4)
        if r == 0:
            p_cp[p].wait()
        _, h = xrow(g)

        x_cp[g].wait()

        @plsc.parallel_loop(0, nvec, unroll=16)
        def _(i):
            rr = i // per_row
            off = pl.multiple_of((i % per_row) * 16, 16)
            plsc.addupdate(bufx.at[s, rr, pl.ds(off, 16)],
                           bufp[p & 1, h * _XCH + rr, pl.ds(off, 16)])

        if r == xc_per_group - 1 and p + 2 < npos:
            load_p(p + 2)
        row, _ = xrow(g)
        out_cp[g] = pltpu.async_copy(
            bufx.at[s], out_hbm.at[pl.ds(row, _XCH)], so[s])
    for g in range(max(0, nx - 5), nx):
        out_cp[g].wait()


def _sc_add(x, pos_embedding):
    batch, seq, hid = x.shape
    x2 = x.reshape(batch * seq, hid)
    mesh = plsc.VectorSubcoreMesh(core_axis_name="c", subcore_axis_name="s")
    out2 = pl.kernel(
        functools.partial(_sc_body, batch, seq, hid),
        out_type=jax.ShapeDtypeStruct((batch * seq, hid), x.dtype),
        mesh=mesh,
        scratch_types=[
            pltpu.VMEM((5, _XCH, hid), jnp.float32),
            pltpu.VMEM((2, _PCH, hid), jnp.float32),
            pltpu.SemaphoreType.DMA,
            pltpu.SemaphoreType.DMA,
            pltpu.SemaphoreType.DMA,
            pltpu.SemaphoreType.DMA,
            pltpu.SemaphoreType.DMA,
            pltpu.SemaphoreType.DMA,
            pltpu.SemaphoreType.DMA,
            pltpu.SemaphoreType.DMA,
            pltpu.SemaphoreType.DMA,
            pltpu.SemaphoreType.DMA,
            pltpu.SemaphoreType.DMA,
            pltpu.SemaphoreType.DMA,
        ],
    )(x2, pos_embedding)
    return out2.reshape(batch, seq, hid)


def kernel(x, pos_embedding):
    return _sc_add(x, pos_embedding)


# speedup vs baseline: 1.1486x; 1.1486x over previous
import jax
import jax.numpy as jnp
from jax import lax
from jax.experimental import pallas as pl
from jax.experimental.pallas import tpu as pltpu
from jax.experimental.pallas import tpu_sc as plsc
import functools

_NC = 2
_NS = 16
_NW = _NC * _NS
_XCH = 16
_PCH = 16


def _sc_body(batch, seq, hid, x_hbm, pos_hbm, out_hbm,
             bufx, bufp, sx0, sx1, sx2, sx3, sx4, sp0, sp1, so0, so1, so2, so3, so4):
    cid = lax.axis_index("c")
    sid = lax.axis_index("s")
    wid = sid * _NC + cid
    span = seq // _NW
    pstart = wid * span
    npos = span // _PCH
    xc_per_group = (_PCH // _XCH) * batch
    nx = npos * xc_per_group
    per_row = hid // 16
    nvec = _XCH * per_row
    sx = (sx0, sx1, sx2, sx3, sx4)
    sp = (sp0, sp1)
    so = (so0, so1, so2, so3, so4)

    def xrow(g):
        p, r = divmod(g, xc_per_group)
        h, b = divmod(r, batch)
        return b * seq + pstart + p * _PCH + h * _XCH, h

    x_cp = [None] * nx
    p_cp = [None] * npos
    out_cp = [None] * nx

    def load_x(g):
        row, _ = xrow(g)
        x_cp[g] = pltpu.async_copy(
            x_hbm.at[pl.ds(row, _XCH)], bufx.at[g % 5], sx[g % 5])

    def load_p(p):
        p_cp[p] = pltpu.async_copy(
            pos_hbm.at[pl.ds(pstart + p * _PCH, _PCH)], bufp.at[p & 1],
            sp[p & 1])

    load_p(0)
    if npos > 1:
        load_p(1)
    for g0 in range(min(3, nx)):
        load_x(g0)
    for g in range(nx):
        s = g % 5
        p, r = divmod(g, xc_per_group)
        if g + 3 < nx:
            if g >= 2:
                out_cp[g - 2].wait()
            load_x(g + 3)
        if r == 0:
            p_cp[p].wait()
        _, h = xrow(g)

        x_cp[g].wait()

        @plsc.parallel_loop(0, nvec, unroll=8)
        def _(i):
            rr = i // per_row
            off = pl.multiple_of((i % per_row) * 16, 16)
            plsc.addupdate(bufx.at[s, rr, pl.ds(off, 16)],
                           bufp[p & 1, h * _XCH + rr, pl.ds(off, 16)])

        if r == xc_per_group - 1 and p + 2 < npos:
            load_p(p + 2)
        row, _ = xrow(g)
        out_cp[g] = pltpu.async_copy(
            bufx.at[s], out_hbm.at[pl.ds(row, _XCH)], so[s])
    for g in range(max(0, nx - 5), nx):
        out_cp[g].wait()


def _sc_add(x, pos_embedding):
    batch, seq, hid = x.shape
    x2 = x.reshape(batch * seq, hid)
    mesh = plsc.VectorSubcoreMesh(core_axis_name="c", subcore_axis_name="s")
    out2 = pl.kernel(
        functools.partial(_sc_body, batch, seq, hid),
        out_type=jax.ShapeDtypeStruct((batch * seq, hid), x.dtype),
        mesh=mesh,
        scratch_types=[
            pltpu.VMEM((5, _XCH, hid), jnp.float32),
            pltpu.VMEM((2, _PCH, hid), jnp.float32),
            pltpu.SemaphoreType.DMA,
            pltpu.SemaphoreType.DMA,
            pltpu.SemaphoreType.DMA,
            pltpu.SemaphoreType.DMA,
            pltpu.SemaphoreType.DMA,
            pltpu.SemaphoreType.DMA,
            pltpu.SemaphoreType.DMA,
            pltpu.SemaphoreType.DMA,
            pltpu.SemaphoreType.DMA,
            pltpu.SemaphoreType.DMA,
            pltpu.SemaphoreType.DMA,
            pltpu.SemaphoreType.DMA,
        ],
    )(x2, pos_embedding)
    return out2.reshape(batch, seq, hid)


def kernel(x, pos_embedding):
    return _sc_add(x, pos_embedding)
